# baseline (speedup 1.0000x reference)
import functools

import jax
import jax.numpy as jnp
import numpy as np
from jax import lax
from jax.experimental import pallas as pl
from jax.experimental.pallas import tpu as pltpu
from jax.experimental.pallas import tpu_sc as plsc

D_SIZES = (128, 256, 128, 512)
D_OFFS = (0, 128, 384, 512)
D_SUM = 1024
N_OUT = 1024
B_TOK = 16 * 2048

PACKED = (False, True, False, True)
G_WIDTH = (128, 128, 128, 256)
G_DTYPE = (jnp.float32, jnp.int32, jnp.float32, jnp.int32)

NUM_CORES = 2
NUM_SUBCORES = 16
NUM_WORKERS = NUM_CORES * NUM_SUBCORES

N_SLICES = 4
H_TOK = B_TOK // N_SLICES
TOK_PER_W = H_TOK // NUM_WORKERS
CHUNK = min(128, TOK_PER_W)

BLK = 1024

_PERM = np.concatenate([
    np.concatenate([np.arange(o, o + d, 2), np.arange(o + 1, o + d, 2)])
    if p else np.arange(o, o + d)
    for o, d, p in zip(D_OFFS, D_SIZES, PACKED)
])


def _sc_gather(xT, tp0, tp1, tp2, tp3, sbase):
    mesh = plsc.VectorSubcoreMesh(core_axis_name="c", subcore_axis_name="s")
    out_type = tuple(
        jax.ShapeDtypeStruct((H_TOK, w), dt)
        for w, dt in zip(G_WIDTH, G_DTYPE)
    )
    scratch = (
        [pltpu.VMEM((TOK_PER_W,), jnp.int32) for _ in range(4)]
        + [pltpu.VMEM((CHUNK, G_WIDTH[i]), G_DTYPE[i]) for i in range(3)]
        + [pltpu.VMEM((CHUNK, G_WIDTH[3]), G_DTYPE[3]) for _ in range(2)]
        + [pltpu.SemaphoreType.DMA, pltpu.SemaphoreType.DMA]
    )

    @functools.partial(
        pl.kernel, mesh=mesh, out_type=out_type, scratch_types=scratch
    )
    def k(x_hbm, t0_hbm, t1_hbm, t2_hbm, t3_hbm, g0, g1, g2, g3,
          i0, i1, i2, i3, r0, r1, r2, r3a, r3b, sem_g, sem_w):
        wid = lax.axis_index("s") * NUM_CORES + lax.axis_index("c")
        base = wid * TOK_PER_W
        tabs = (t0_hbm, t1_hbm, t2_hbm, t3_hbm)
        gouts = (g0, g1, g2, g3)
        idxs = (i0, i1, i2, i3)
        for i in range(4):
            pltpu.sync_copy(
                x_hbm.at[i, pl.ds(sbase + base, TOK_PER_W)], idxs[i]
            )

        def bufof(i, c):
            if i == 3:
                return (r3a, r3b)[c % 2]
            return (r0, r1, r2)[i]

        jobs = []
        nch = TOK_PER_W // CHUNK
        for c in range(nch):
            for i in range(4):
                jobs.append((i, c))

        fifo = []
        prev = None
        for i, c in jobs:
            buf = bufof(i, c)
            if any(b is buf for _, b in fifo):
                while True:
                    h, b = fifo.pop(0)
                    h.wait()
                    if b is buf:
                        break
            gh = pltpu.async_copy(
                tabs[i].at[idxs[i].at[pl.ds(c * CHUNK, CHUNK)]],
                buf, sem_g,
            )
            if prev is not None:
                pgh, pi, pc, pbuf = prev
                pgh.wait()
                wh = pltpu.async_copy(
                    pbuf,
                    gouts[pi].at[pl.ds(base + pc * CHUNK, CHUNK)],
                    sem_w,
                )
                fifo.append((wh, pbuf))
            prev = (gh, i, c, buf)
        pgh, pi, pc, pbuf = prev
        pgh.wait()
        wh = pltpu.async_copy(
            pbuf, gouts[pi].at[pl.ds(base + pc * CHUNK, CHUNK)],
            sem_w,
        )
        fifo.append((wh, pbuf))
        for h, _ in fifo:
            h.wait()

    return k(xT, tp0, tp1, tp2, tp3)


def _matmul_body(g0, g1, g2, g3, w, bb, o):
    blocks = (g0, g1, g2, g3)
    acc = bb[...].astype(jnp.float32)
    roff = 0
    for i in range(4):
        x = blocks[i][...]
        if PACKED[i]:
            lo = lax.bitcast_convert_type(
                jnp.left_shift(x, 16), jnp.float32
            ).astype(jnp.bfloat16)
            hi = lax.bitcast_convert_type(
                jnp.bitwise_and(x, jnp.int32(-65536)), jnp.float32
            ).astype(jnp.bfloat16)
            half = G_WIDTH[i]
            acc = acc + jnp.dot(
                lo, w[roff:roff + half, :],
                preferred_element_type=jnp.float32,
            )
            acc = acc + jnp.dot(
                hi, w[roff + half:roff + 2 * half, :],
                preferred_element_type=jnp.float32,
            )
            roff += 2 * half
        else:
            acc = acc + jnp.dot(
                x.astype(jnp.bfloat16), w[roff:roff + G_WIDTH[i], :],
                preferred_element_type=jnp.float32,
            )
            roff += G_WIDTH[i]
    o[...] = acc


def _tc_matmul_slice(acc, gs, W, b2, s):
    blk0 = s * H_TOK // BLK
    grid = (H_TOK // BLK,)
    in_specs = [
        pl.BlockSpec(memory_space=pl.ANY),
    ] + [
        pl.BlockSpec((BLK, w), lambda i: (i, 0)) for w in G_WIDTH
    ] + [
        pl.BlockSpec((D_SUM, N_OUT), lambda i: (0, 0)),
        pl.BlockSpec((1, N_OUT), lambda i: (0, 0)),
    ]
    out_specs = pl.BlockSpec((BLK, N_OUT), lambda i: (i + blk0, 0))

    def body(a, g0, g1, g2, g3, w, bb, o):
        del a
        _matmul_body(g0, g1, g2, g3, w, bb, o)

    return pl.pallas_call(
        body,
        grid=grid,
        in_specs=in_specs,
        out_specs=out_specs,
        out_shape=jax.ShapeDtypeStruct((B_TOK, N_OUT), jnp.float32),
        input_output_aliases={0: 0},
    )(acc, *gs, W, b2)


def _tc_matmul_first(gs, W, b2):
    grid = (H_TOK // BLK,)
    in_specs = [
        pl.BlockSpec((BLK, w), lambda i: (i, 0)) for w in G_WIDTH
    ] + [
        pl.BlockSpec((D_SUM, N_OUT), lambda i: (0, 0)),
        pl.BlockSpec((1, N_OUT), lambda i: (0, 0)),
    ]
    out_specs = pl.BlockSpec((BLK, N_OUT), lambda i: (i, 0))

    return pl.pallas_call(
        _matmul_body,
        grid=grid,
        in_specs=in_specs,
        out_specs=out_specs,
        out_shape=jax.ShapeDtypeStruct((B_TOK, N_OUT), jnp.float32),
    )(*gs, W, b2)


def _pack(t):
    tb = t.astype(jnp.bfloat16)
    return lax.bitcast_convert_type(
        tb.reshape(t.shape[0], t.shape[1] // 2, 2), jnp.int32
    )


def kernel(x, t0, t1, t2, t3, W, b):
    bsz, seq, _ = x.shape
    xT = jnp.transpose(x.reshape(-1, 4).astype(jnp.int32))
    Wr = W.astype(jnp.bfloat16)[_PERM]
    b2 = b.reshape(1, N_OUT)
    tps = (t0, _pack(t1), t2, _pack(t3))
    gs_slices = [
        _sc_gather(xT, *tps, s * H_TOK) for s in range(N_SLICES)
    ]
    out = _tc_matmul_first(gs_slices[0], Wr, b2)
    for s in range(1, N_SLICES):
        out = _tc_matmul_slice(out, gs_slices[s], Wr, b2, s)
    return out.reshape(bsz, seq, N_OUT)


# speedup vs baseline: 2.2631x; 2.2631x over previous
import functools

import jax
import jax.numpy as jnp
from jax import lax
from jax.experimental import pallas as pl
from jax.experimental.pallas import tpu as pltpu
from jax.experimental.pallas import tpu_sc as plsc

D_SIZES = (128, 256, 128, 512)
D_OFFS = (0, 128, 384, 512)
D_SUM = 1024
N_OUT = 1024
B_TOK = 16 * 2048

PACKED = (False, True, False, True)
G_WIDTH = (128, 128, 128, 256)
G_DTYPE = (jnp.float32, jnp.int32, jnp.float32, jnp.int32)

NUM_CORES = 2
NUM_SUBCORES = 16
NUM_WORKERS = NUM_CORES * NUM_SUBCORES

N_SLICES = 4
H_TOK = B_TOK // N_SLICES
TOK_PER_W = H_TOK // NUM_WORKERS
CHUNK = min(128, TOK_PER_W)

BLK = 1024

def _sc_gather(xr, tp0, tp1, tp2, tp3, sbase):
    mesh = plsc.VectorSubcoreMesh(core_axis_name="c", subcore_axis_name="s")
    out_type = tuple(
        jax.ShapeDtypeStruct((H_TOK, w), dt)
        for w, dt in zip(G_WIDTH, G_DTYPE)
    )
    scratch = (
        [pltpu.VMEM((TOK_PER_W,), jnp.int32) for _ in range(4)]
        + [pltpu.VMEM((CHUNK, G_WIDTH[i]), G_DTYPE[i]) for i in range(3)]
        + [pltpu.VMEM((CHUNK, G_WIDTH[3]), G_DTYPE[3]) for _ in range(2)]
        + [pltpu.SemaphoreType.DMA, pltpu.SemaphoreType.DMA]
    )

    @functools.partial(
        pl.kernel, mesh=mesh, out_type=out_type, scratch_types=scratch
    )
    def k(x_hbm, t0_hbm, t1_hbm, t2_hbm, t3_hbm, g0, g1, g2, g3,
          i0, i1, i2, i3, r0, r1, r2, r3a, r3b, sem_g, sem_w):
        wid = lax.axis_index("s") * NUM_CORES + lax.axis_index("c")
        base = wid * TOK_PER_W
        tabs = (t0_hbm, t1_hbm, t2_hbm, t3_hbm)
        gouts = (g0, g1, g2, g3)
        idxs = (i0, i1, i2, i3)
        for i in range(4):
            pltpu.sync_copy(
                x_hbm.at[i, pl.ds(sbase + base, TOK_PER_W)], idxs[i]
            )

        def bufof(i, c):
            if i == 3:
                return (r3a, r3b)[c % 2]
            return (r0, r1, r2)[i]

        jobs = []
        nch = TOK_PER_W // CHUNK
        for c in range(nch):
            for i in range(4):
                jobs.append((i, c))

        fifo = []
        prev = None
        for i, c in jobs:
            buf = bufof(i, c)
            if any(b is buf for _, b in fifo):
                while True:
                    h, b = fifo.pop(0)
                    h.wait()
                    if b is buf:
                        break
            gh = pltpu.async_copy(
                tabs[i].at[idxs[i].at[pl.ds(c * CHUNK, CHUNK)]],
                buf, sem_g,
            )
            if prev is not None:
                pgh, pi, pc, pbuf = prev
                pgh.wait()
                wh = pltpu.async_copy(
                    pbuf,
                    gouts[pi].at[pl.ds(base + pc * CHUNK, CHUNK)],
                    sem_w,
                )
                fifo.append((wh, pbuf))
            prev = (gh, i, c, buf)
        pgh, pi, pc, pbuf = prev
        pgh.wait()
        wh = pltpu.async_copy(
            pbuf, gouts[pi].at[pl.ds(base + pc * CHUNK, CHUNK)],
            sem_w,
        )
        fifo.append((wh, pbuf))
        for h, _ in fifo:
            h.wait()

    return k(xr, tp0, tp1, tp2, tp3)


def _matmul_body(g0, g1, g2, g3, w, bb, o):
    blocks = (g0, g1, g2, g3)
    acc = bb[...].astype(jnp.float32)
    roff = 0
    for i in range(4):
        x = blocks[i][...]
        if PACKED[i]:
            lo = lax.bitcast_convert_type(
                jnp.left_shift(x, 16), jnp.float32
            ).astype(jnp.bfloat16)
            hi = lax.bitcast_convert_type(
                jnp.bitwise_and(x, jnp.int32(-65536)), jnp.float32
            ).astype(jnp.bfloat16)
            half = G_WIDTH[i]
            acc = acc + jnp.dot(
                lo, w[roff:roff + half, :],
                preferred_element_type=jnp.float32,
            )
            acc = acc + jnp.dot(
                hi, w[roff + half:roff + 2 * half, :],
                preferred_element_type=jnp.float32,
            )
            roff += 2 * half
        else:
            acc = acc + jnp.dot(
                x.astype(jnp.bfloat16), w[roff:roff + G_WIDTH[i], :],
                preferred_element_type=jnp.float32,
            )
            roff += G_WIDTH[i]
    o[...] = acc


def _tc_matmul_slice(acc, gs, W, b2, s):
    blk0 = s * H_TOK // BLK
    grid = (H_TOK // BLK,)
    in_specs = [
        pl.BlockSpec(memory_space=pl.ANY),
    ] + [
        pl.BlockSpec((BLK, w), lambda i: (i, 0)) for w in G_WIDTH
    ] + [
        pl.BlockSpec((D_SUM, N_OUT), lambda i: (0, 0)),
        pl.BlockSpec((1, N_OUT), lambda i: (0, 0)),
    ]
    out_specs = pl.BlockSpec((BLK, N_OUT), lambda i: (i + blk0, 0))

    def body(a, g0, g1, g2, g3, w, bb, o):
        del a
        _matmul_body(g0, g1, g2, g3, w, bb, o)

    return pl.pallas_call(
        body,
        grid=grid,
        in_specs=in_specs,
        out_specs=out_specs,
        out_shape=jax.ShapeDtypeStruct((B_TOK, N_OUT), jnp.float32),
        input_output_aliases={0: 0},
    )(acc, *gs, W, b2)


def _tc_matmul_first(gs, W, b2):
    grid = (H_TOK // BLK,)
    in_specs = [
        pl.BlockSpec((BLK, w), lambda i: (i, 0)) for w in G_WIDTH
    ] + [
        pl.BlockSpec((D_SUM, N_OUT), lambda i: (0, 0)),
        pl.BlockSpec((1, N_OUT), lambda i: (0, 0)),
    ]
    out_specs = pl.BlockSpec((BLK, N_OUT), lambda i: (i, 0))

    return pl.pallas_call(
        _matmul_body,
        grid=grid,
        in_specs=in_specs,
        out_specs=out_specs,
        out_shape=jax.ShapeDtypeStruct((B_TOK, N_OUT), jnp.float32),
    )(*gs, W, b2)


def _pack(t):
    rows, d = t.shape
    half = d // 2
    brow = 1000

    def body(tref, oref):
        a16 = lax.bitcast_convert_type(
            tref[:, :half].astype(jnp.bfloat16), jnp.uint16
        ).astype(jnp.uint32)
        b16 = lax.bitcast_convert_type(
            tref[:, half:].astype(jnp.bfloat16), jnp.uint16
        ).astype(jnp.uint32)
        oref[...] = lax.bitcast_convert_type(
            a16 | jnp.left_shift(b16, 16), jnp.int32
        )

    return pl.pallas_call(
        body,
        grid=(rows // brow,),
        in_specs=[pl.BlockSpec((brow, d), lambda i: (i, 0))],
        out_specs=pl.BlockSpec((brow, half), lambda i: (i, 0)),
        out_shape=jax.ShapeDtypeStruct((rows, half), jnp.int32),
    )(t)


def kernel(x, t0, t1, t2, t3, W, b):
    bsz, seq, _ = x.shape
    xr = jnp.transpose(x.reshape(-1, 4).astype(jnp.int32))
    Wr = W.astype(jnp.bfloat16)
    b2 = b.reshape(1, N_OUT)
    tps = (t0, _pack(t1), t2, _pack(t3))
    gs_slices = [
        _sc_gather(xr, *tps, s * H_TOK) for s in range(N_SLICES)
    ]
    out = _tc_matmul_first(gs_slices[0], Wr, b2)
    for s in range(1, N_SLICES):
        out = _tc_matmul_slice(out, gs_slices[s], Wr, b2, s)
    return out.reshape(bsz, seq, N_OUT)


# speedup vs baseline: 2.3024x; 1.0173x over previous
import functools

import jax
import jax.numpy as jnp
from jax import lax
from jax.experimental import pallas as pl
from jax.experimental.pallas import tpu as pltpu
from jax.experimental.pallas import tpu_sc as plsc

D_SIZES = (128, 256, 128, 512)
D_OFFS = (0, 128, 384, 512)
D_SUM = 1024
N_OUT = 1024
B_TOK = 16 * 2048

PACKED = (False, True, False, True)
G_WIDTH = (128, 128, 128, 256)
G_DTYPE = (jnp.float32, jnp.int32, jnp.float32, jnp.int32)

NUM_CORES = 2
NUM_SUBCORES = 16
NUM_WORKERS = NUM_CORES * NUM_SUBCORES

N_SLICES = 2
H_TOK = B_TOK // N_SLICES
TOK_PER_W = H_TOK // NUM_WORKERS
CHUNK = min(128, TOK_PER_W)

BLK = 1024

def _sc_gather(xr, tp0, tp1, tp2, tp3, sbase):
    mesh = plsc.VectorSubcoreMesh(core_axis_name="c", subcore_axis_name="s")
    out_type = tuple(
        jax.ShapeDtypeStruct((H_TOK, w), dt)
        for w, dt in zip(G_WIDTH, G_DTYPE)
    )
    scratch = (
        [pltpu.VMEM((TOK_PER_W,), jnp.int32) for _ in range(4)]
        + [pltpu.VMEM((CHUNK, G_WIDTH[i]), G_DTYPE[i]) for i in range(3)]
        + [pltpu.VMEM((CHUNK, G_WIDTH[3]), G_DTYPE[3]) for _ in range(2)]
        + [pltpu.SemaphoreType.DMA, pltpu.SemaphoreType.DMA]
    )

    @functools.partial(
        pl.kernel, mesh=mesh, out_type=out_type, scratch_types=scratch
    )
    def k(x_hbm, t0_hbm, t1_hbm, t2_hbm, t3_hbm, g0, g1, g2, g3,
          i0, i1, i2, i3, r0, r1, r2, r3a, r3b, sem_g, sem_w):
        wid = lax.axis_index("s") * NUM_CORES + lax.axis_index("c")
        base = wid * TOK_PER_W
        tabs = (t0_hbm, t1_hbm, t2_hbm, t3_hbm)
        gouts = (g0, g1, g2, g3)
        idxs = (i0, i1, i2, i3)
        for i in range(4):
            pltpu.sync_copy(
                x_hbm.at[i, pl.ds(sbase + base, TOK_PER_W)], idxs[i]
            )

        def bufof(i, c):
            if i == 3:
                return (r3a, r3b)[c % 2]
            return (r0, r1, r2)[i]

        jobs = []
        nch = TOK_PER_W // CHUNK
        for c in range(nch):
            for i in range(4):
                jobs.append((i, c))

        fifo = []
        prev = None
        for i, c in jobs:
            buf = bufof(i, c)
            if any(b is buf for _, b in fifo):
                while True:
                    h, b = fifo.pop(0)
                    h.wait()
                    if b is buf:
                        break
            gh = pltpu.async_copy(
                tabs[i].at[idxs[i].at[pl.ds(c * CHUNK, CHUNK)]],
                buf, sem_g,
            )
            if prev is not None:
                pgh, pi, pc, pbuf = prev
                pgh.wait()
                wh = pltpu.async_copy(
                    pbuf,
                    gouts[pi].at[pl.ds(base + pc * CHUNK, CHUNK)],
                    sem_w,
                )
                fifo.append((wh, pbuf))
            prev = (gh, i, c, buf)
        pgh, pi, pc, pbuf = prev
        pgh.wait()
        wh = pltpu.async_copy(
            pbuf, gouts[pi].at[pl.ds(base + pc * CHUNK, CHUNK)],
            sem_w,
        )
        fifo.append((wh, pbuf))
        for h, _ in fifo:
            h.wait()

    return k(xr, tp0, tp1, tp2, tp3)


def _matmul_body(g0, g1, g2, g3, w, bb, o):
    blocks = (g0, g1, g2, g3)
    acc = bb[...].astype(jnp.float32)
    roff = 0
    for i in range(4):
        x = blocks[i][...]
        if PACKED[i]:
            lo = lax.bitcast_convert_type(
                jnp.left_shift(x, 16), jnp.float32
            ).astype(jnp.bfloat16)
            hi = lax.bitcast_convert_type(
                jnp.bitwise_and(x, jnp.int32(-65536)), jnp.float32
            ).astype(jnp.bfloat16)
            half = G_WIDTH[i]
            acc = acc + jnp.dot(
                lo, w[roff:roff + half, :],
                preferred_element_type=jnp.float32,
            )
            acc = acc + jnp.dot(
                hi, w[roff + half:roff + 2 * half, :],
                preferred_element_type=jnp.float32,
            )
            roff += 2 * half
        else:
            acc = acc + jnp.dot(
                x.astype(jnp.bfloat16), w[roff:roff + G_WIDTH[i], :],
                preferred_element_type=jnp.float32,
            )
            roff += G_WIDTH[i]
    o[...] = acc


def _tc_matmul_slice(acc, gs, W, b2, s):
    blk0 = s * H_TOK // BLK
    grid = (H_TOK // BLK,)
    in_specs = [
        pl.BlockSpec(memory_space=pl.ANY),
    ] + [
        pl.BlockSpec((BLK, w), lambda i: (i, 0)) for w in G_WIDTH
    ] + [
        pl.BlockSpec((D_SUM, N_OUT), lambda i: (0, 0)),
        pl.BlockSpec((1, N_OUT), lambda i: (0, 0)),
    ]
    out_specs = pl.BlockSpec((BLK, N_OUT), lambda i: (i + blk0, 0))

    def body(a, g0, g1, g2, g3, w, bb, o):
        del a
        _matmul_body(g0, g1, g2, g3, w, bb, o)

    return pl.pallas_call(
        body,
        grid=grid,
        in_specs=in_specs,
        out_specs=out_specs,
        out_shape=jax.ShapeDtypeStruct((B_TOK, N_OUT), jnp.float32),
        input_output_aliases={0: 0},
    )(acc, *gs, W, b2)


def _tc_matmul_first(gs, W, b2):
    grid = (H_TOK // BLK,)
    in_specs = [
        pl.BlockSpec((BLK, w), lambda i: (i, 0)) for w in G_WIDTH
    ] + [
        pl.BlockSpec((D_SUM, N_OUT), lambda i: (0, 0)),
        pl.BlockSpec((1, N_OUT), lambda i: (0, 0)),
    ]
    out_specs = pl.BlockSpec((BLK, N_OUT), lambda i: (i, 0))

    return pl.pallas_call(
        _matmul_body,
        grid=grid,
        in_specs=in_specs,
        out_specs=out_specs,
        out_shape=jax.ShapeDtypeStruct((B_TOK, N_OUT), jnp.float32),
    )(*gs, W, b2)


def _pack(t):
    rows, d = t.shape
    half = d // 2
    brow = 1000

    def body(tref, oref):
        a16 = lax.bitcast_convert_type(
            tref[:, :half].astype(jnp.bfloat16), jnp.uint16
        ).astype(jnp.uint32)
        b16 = lax.bitcast_convert_type(
            tref[:, half:].astype(jnp.bfloat16), jnp.uint16
        ).astype(jnp.uint32)
        oref[...] = lax.bitcast_convert_type(
            a16 | jnp.left_shift(b16, 16), jnp.int32
        )

    return pl.pallas_call(
        body,
        grid=(rows // brow,),
        in_specs=[pl.BlockSpec((brow, d), lambda i: (i, 0))],
        out_specs=pl.BlockSpec((brow, half), lambda i: (i, 0)),
        out_shape=jax.ShapeDtypeStruct((rows, half), jnp.int32),
    )(t)


def kernel(x, t0, t1, t2, t3, W, b):
    bsz, seq, _ = x.shape
    xr = jnp.transpose(x.reshape(-1, 4).astype(jnp.int32))
    Wr = W.astype(jnp.bfloat16)
    b2 = b.reshape(1, N_OUT)
    tps = (t0, _pack(t1), t2, _pack(t3))
    gs_slices = [
        _sc_gather(xr, *tps, s * H_TOK) for s in range(N_SLICES)
    ]
    out = _tc_matmul_first(gs_slices[0], Wr, b2)
    for s in range(1, N_SLICES):
        out = _tc_matmul_slice(out, gs_slices[s], Wr, b2, s)
    return out.reshape(bsz, seq, N_OUT)


# speedup vs baseline: 2.3138x; 1.0049x over previous
import functools

import jax
import jax.numpy as jnp
from jax import lax
from jax.experimental import pallas as pl
from jax.experimental.pallas import tpu as pltpu
from jax.experimental.pallas import tpu_sc as plsc

D_SIZES = (128, 256, 128, 512)
D_OFFS = (0, 128, 384, 512)
D_SUM = 1024
N_OUT = 1024
B_TOK = 16 * 2048

PACKED = (False, True, False, True)
G_WIDTH = (128, 128, 128, 256)
G_DTYPE = (jnp.float32, jnp.int32, jnp.float32, jnp.int32)

NUM_CORES = 2
NUM_SUBCORES = 16
NUM_WORKERS = NUM_CORES * NUM_SUBCORES

N_SLICES = 2
H_TOK = B_TOK // N_SLICES
TOK_PER_W = H_TOK // NUM_WORKERS
CHUNK = min(128, TOK_PER_W)

BLK = 1024

def _sc_gather(xr, tp0, tp1, tp2, tp3, sbase):
    mesh = plsc.VectorSubcoreMesh(core_axis_name="c", subcore_axis_name="s")
    out_type = tuple(
        jax.ShapeDtypeStruct((H_TOK, w), dt)
        for w, dt in zip(G_WIDTH, G_DTYPE)
    )
    scratch = (
        [pltpu.VMEM((4, TOK_PER_W), jnp.int32)]
        + [pltpu.VMEM((CHUNK, G_WIDTH[i]), G_DTYPE[i]) for i in range(3)]
        + [pltpu.VMEM((CHUNK, G_WIDTH[3]), G_DTYPE[3]) for _ in range(2)]
        + [pltpu.SemaphoreType.DMA, pltpu.SemaphoreType.DMA]
    )

    @functools.partial(
        pl.kernel, mesh=mesh, out_type=out_type, scratch_types=scratch
    )
    def k(x_hbm, t0_hbm, t1_hbm, t2_hbm, t3_hbm, g0, g1, g2, g3,
          idx2, r0, r1, r2, r3a, r3b, sem_g, sem_w):
        wid = lax.axis_index("s") * NUM_CORES + lax.axis_index("c")
        base = wid * TOK_PER_W
        tabs = (t0_hbm, t1_hbm, t2_hbm, t3_hbm)
        gouts = (g0, g1, g2, g3)
        pltpu.sync_copy(
            x_hbm.at[:, pl.ds(sbase + base, TOK_PER_W)], idx2
        )

        def bufof(i, c):
            if i == 3:
                return (r3a, r3b)[c % 2]
            return (r0, r1, r2)[i]

        jobs = []
        nch = TOK_PER_W // CHUNK
        for c in range(nch):
            for i in range(4):
                jobs.append((i, c))

        fifo = []
        prev = None
        for i, c in jobs:
            buf = bufof(i, c)
            if any(b is buf for _, b in fifo):
                while True:
                    h, b = fifo.pop(0)
                    h.wait()
                    if b is buf:
                        break
            gh = pltpu.async_copy(
                tabs[i].at[idx2.at[i, pl.ds(c * CHUNK, CHUNK)]],
                buf, sem_g,
            )
            if prev is not None:
                pgh, pi, pc, pbuf = prev
                pgh.wait()
                wh = pltpu.async_copy(
                    pbuf,
                    gouts[pi].at[pl.ds(base + pc * CHUNK, CHUNK)],
                    sem_w,
                )
                fifo.append((wh, pbuf))
            prev = (gh, i, c, buf)
        pgh, pi, pc, pbuf = prev
        pgh.wait()
        wh = pltpu.async_copy(
            pbuf, gouts[pi].at[pl.ds(base + pc * CHUNK, CHUNK)],
            sem_w,
        )
        fifo.append((wh, pbuf))
        for h, _ in fifo:
            h.wait()

    return k(xr, tp0, tp1, tp2, tp3)


def _matmul_body(g0, g1, g2, g3, w, bb, o):
    blocks = (g0, g1, g2, g3)
    acc = bb[...].astype(jnp.float32)
    roff = 0
    for i in range(4):
        x = blocks[i][...]
        if PACKED[i]:
            lo = lax.bitcast_convert_type(
                jnp.left_shift(x, 16), jnp.float32
            ).astype(jnp.bfloat16)
            hi = lax.bitcast_convert_type(
                jnp.bitwise_and(x, jnp.int32(-65536)), jnp.float32
            ).astype(jnp.bfloat16)
            half = G_WIDTH[i]
            acc = acc + jnp.dot(
                lo, w[roff:roff + half, :],
                preferred_element_type=jnp.float32,
            )
            acc = acc + jnp.dot(
                hi, w[roff + half:roff + 2 * half, :],
                preferred_element_type=jnp.float32,
            )
            roff += 2 * half
        else:
            acc = acc + jnp.dot(
                x.astype(jnp.bfloat16), w[roff:roff + G_WIDTH[i], :],
                preferred_element_type=jnp.float32,
            )
            roff += G_WIDTH[i]
    o[...] = acc


def _tc_matmul_slice(acc, gs, W, b2, s):
    blk0 = s * H_TOK // BLK
    grid = (H_TOK // BLK,)
    in_specs = [
        pl.BlockSpec(memory_space=pl.ANY),
    ] + [
        pl.BlockSpec((BLK, w), lambda i: (i, 0)) for w in G_WIDTH
    ] + [
        pl.BlockSpec((D_SUM, N_OUT), lambda i: (0, 0)),
        pl.BlockSpec((1, N_OUT), lambda i: (0, 0)),
    ]
    out_specs = pl.BlockSpec((BLK, N_OUT), lambda i: (i + blk0, 0))

    def body(a, g0, g1, g2, g3, w, bb, o):
        del a
        _matmul_body(g0, g1, g2, g3, w, bb, o)

    return pl.pallas_call(
        body,
        grid=grid,
        in_specs=in_specs,
        out_specs=out_specs,
        out_shape=jax.ShapeDtypeStruct((B_TOK, N_OUT), jnp.float32),
        input_output_aliases={0: 0},
    )(acc, *gs, W, b2)


def _tc_matmul_first(gs, W, b2):
    grid = (H_TOK // BLK,)
    in_specs = [
        pl.BlockSpec((BLK, w), lambda i: (i, 0)) for w in G_WIDTH
    ] + [
        pl.BlockSpec((D_SUM, N_OUT), lambda i: (0, 0)),
        pl.BlockSpec((1, N_OUT), lambda i: (0, 0)),
    ]
    out_specs = pl.BlockSpec((BLK, N_OUT), lambda i: (i, 0))

    return pl.pallas_call(
        _matmul_body,
        grid=grid,
        in_specs=in_specs,
        out_specs=out_specs,
        out_shape=jax.ShapeDtypeStruct((B_TOK, N_OUT), jnp.float32),
    )(*gs, W, b2)


def _pack(t):
    rows, d = t.shape
    half = d // 2
    brow = 1000

    def body(tref, oref):
        a16 = lax.bitcast_convert_type(
            tref[:, :half].astype(jnp.bfloat16), jnp.uint16
        ).astype(jnp.uint32)
        b16 = lax.bitcast_convert_type(
            tref[:, half:].astype(jnp.bfloat16), jnp.uint16
        ).astype(jnp.uint32)
        oref[...] = lax.bitcast_convert_type(
            a16 | jnp.left_shift(b16, 16), jnp.int32
        )

    return pl.pallas_call(
        body,
        grid=(rows // brow,),
        in_specs=[pl.BlockSpec((brow, d), lambda i: (i, 0))],
        out_specs=pl.BlockSpec((brow, half), lambda i: (i, 0)),
        out_shape=jax.ShapeDtypeStruct((rows, half), jnp.int32),
    )(t)


def kernel(x, t0, t1, t2, t3, W, b):
    bsz, seq, _ = x.shape
    xr = jnp.transpose(x.reshape(-1, 4).astype(jnp.int32))
    Wr = W.astype(jnp.bfloat16)
    b2 = b.reshape(1, N_OUT)
    tps = (t0, _pack(t1), t2, _pack(t3))
    gs_slices = [
        _sc_gather(xr, *tps, s * H_TOK) for s in range(N_SLICES)
    ]
    out = _tc_matmul_first(gs_slices[0], Wr, b2)
    for s in range(1, N_SLICES):
        out = _tc_matmul_slice(out, gs_slices[s], Wr, b2, s)
    return out.reshape(bsz, seq, N_OUT)
